# Initial kernel scaffold; baseline (speedup 1.0000x reference)
import jax
import jax.numpy as jnp
from jax.experimental import pallas as pl


def kernel(x, edge_index, h, Wz, bz, Wr, br, Wh, bh, Lz, lbz, Lr, lbr, Lh, lbh, Wo, bo):
    raise NotImplementedError("write your pallas kernel here")


# speedup vs baseline: 26.3692x; 26.3692x over previous
import functools

import jax
import jax.numpy as jnp
from jax import lax
from jax.experimental import pallas as pl
from jax.experimental.pallas import tpu as pltpu
from jax.experimental.pallas import tpu_sc as plsc

N = 10000
E = 320000
D = 128
NC = 2
NS = 16
NW = NC * NS
CH = 80


def _degree_hist(dst, ones_c, zeros_d):
    per_tile = E // NW
    chunks = per_tile // CH
    mesh = plsc.VectorSubcoreMesh(core_axis_name="c", subcore_axis_name="s")

    @functools.partial(
        pl.kernel,
        out_type=jax.ShapeDtypeStruct((NC, N, 1), jnp.float32),
        mesh=mesh,
        scratch_types=[
            pltpu.VMEM((1, CH), jnp.int32),
            pltpu.VMEM((CH, 1), jnp.float32),
            pltpu.VMEM_SHARED((N, 1), jnp.float32),
        ],
    )
    def k(dst_hbm, ones_hbm, zeros_hbm, dp_hbm, didx, ones_v, dsp):
        cid = lax.axis_index("c")
        sid = lax.axis_index("s")

        @pl.when(sid == 0)
        def _():
            pltpu.sync_copy(zeros_hbm, dsp)

        pltpu.sync_copy(ones_hbm, ones_v)
        plsc.subcore_barrier()

        tile_base = (cid * NS + sid) * per_tile

        def body(j, carry):
            base = pl.multiple_of(tile_base + j * CH, 8)
            pltpu.sync_copy(dst_hbm.at[pl.ds(base, CH)], didx.at[0])
            pltpu.sync_copy(ones_v, dsp.at[didx.at[0]], add=True)
            return carry

        lax.fori_loop(0, chunks, body, 0)
        plsc.subcore_barrier()

        @pl.when(sid == 0)
        def _():
            pltpu.sync_copy(dsp, dp_hbm.at[cid])

    return k(dst, ones_c, zeros_d)


def _scale_rows(dp, x):
    BN = 1000
    grid = N // BN

    def body(dp_ref, x_ref, dis_ref, y_ref):
        deg = dp_ref[0] + dp_ref[1] + 1.0
        dis = lax.rsqrt(deg)
        dis_ref[...] = dis
        y_ref[...] = x_ref[...] * dis

    return pl.pallas_call(
        body,
        grid=(grid,),
        in_specs=[
            pl.BlockSpec((NC, BN, 1), lambda i: (0, i, 0)),
            pl.BlockSpec((BN, D), lambda i: (i, 0)),
        ],
        out_specs=[
            pl.BlockSpec((BN, 1), lambda i: (i, 0)),
            pl.BlockSpec((BN, D), lambda i: (i, 0)),
        ],
        out_shape=[
            jax.ShapeDtypeStruct((N, 1), jnp.float32),
            jax.ShapeDtypeStruct((N, D), jnp.float32),
        ],
    )(dp, x)


def _aggregate(srcs, dsts, y, zeros_z):
    per_tile = E // NW
    chunks = per_tile // CH
    rpt = 624
    tail = N - NS * rpt
    mesh = plsc.VectorSubcoreMesh(core_axis_name="c", subcore_axis_name="s")

    @functools.partial(
        pl.kernel,
        out_type=jax.ShapeDtypeStruct((NC, N, D), jnp.float32),
        mesh=mesh,
        scratch_types=[
            pltpu.VMEM((CH,), jnp.int32),
            pltpu.VMEM((1, CH), jnp.int32),
            pltpu.VMEM((CH, D), jnp.float32),
            pltpu.VMEM_SHARED((N, D), jnp.float32),
            pltpu.SemaphoreType.DMA,
        ],
    )
    def k(src_hbm, dst_hbm, y_hbm, zeros_hbm, zp_hbm, sidx, didx, rows, zsp, sem):
        cid = lax.axis_index("c")
        sid = lax.axis_index("s")
        r0 = pl.multiple_of(sid * rpt, 8)
        pltpu.sync_copy(zeros_hbm.at[pl.ds(r0, rpt)], zsp.at[pl.ds(r0, rpt)])

        @pl.when(sid == NS - 1)
        def _():
            pltpu.sync_copy(zeros_hbm.at[pl.ds(NS * rpt, tail)],
                            zsp.at[pl.ds(NS * rpt, tail)])

        plsc.subcore_barrier()

        tile_base = (cid * NS + sid) * per_tile

        def body(j, carry):
            base = pl.multiple_of(tile_base + j * CH, 8)
            pltpu.sync_copy(src_hbm.at[pl.ds(base, CH)], sidx)
            pltpu.sync_copy(dst_hbm.at[pl.ds(base, CH)], didx.at[0])
            pltpu.async_copy(y_hbm.at[sidx], rows, sem).wait()
            pltpu.sync_copy(rows, zsp.at[didx.at[0]], add=True)
            return carry

        lax.fori_loop(0, chunks, body, 0)
        plsc.subcore_barrier()
        pltpu.sync_copy(zsp.at[pl.ds(r0, rpt)],
                        zp_hbm.at[cid].at[pl.ds(r0, rpt)])

        @pl.when(sid == NS - 1)
        def _():
            pltpu.sync_copy(zsp.at[pl.ds(NS * rpt, tail)],
                            zp_hbm.at[cid].at[pl.ds(NS * rpt, tail)])

    return k(srcs, dsts, y, zeros_z)


def _dense_cell(zp, y, dis, h, Wz, bz2, Wr, br2, Wh, bh2,
                Lz, lbz2, Lr, lbr2, Lh, lbh2, Wo, bo2):
    BN = 1000
    grid = N // BN
    f32 = jnp.float32

    def body(zp_ref, y_ref, dis_ref, h_ref, Wz_ref, bz_ref, Wr_ref, br_ref,
             Wh_ref, bh_ref, Lz_ref, lbz_ref, Lr_ref, lbr_ref, Lh_ref,
             lbh_ref, Wo_ref, bo_ref, out_ref, hn_ref):
        z = zp_ref[0] + zp_ref[1] + y_ref[...]
        agg = dis_ref[...] * z
        h_blk = h_ref[...]
        Lz_m = Lz_ref[...]
        Lr_m = Lr_ref[...]
        Lh_m = Lh_ref[...]

        def gate(W_ref, b_ref, L, lb_ref, hh):
            A = jnp.dot(W_ref[...], L[:D], preferred_element_type=f32)
            cb = jnp.dot(b_ref[...], L[:D], preferred_element_type=f32) + lb_ref[...]
            return (jnp.dot(agg, A, preferred_element_type=f32)
                    + jnp.dot(hh, L[D:], preferred_element_type=f32) + cb)

        Zg = jax.nn.sigmoid(gate(Wz_ref, bz_ref, Lz_m, lbz_ref, h_blk))
        Rg = jax.nn.sigmoid(gate(Wr_ref, br_ref, Lr_m, lbr_ref, h_blk))
        Ht = jnp.tanh(gate(Wh_ref, bh_ref, Lh_m, lbh_ref, h_blk * Rg))
        hn = Zg * h_blk + (1.0 - Zg) * Ht
        hn_ref[...] = hn
        out_ref[...] = jax.nn.sigmoid(
            jnp.dot(hn, Wo_ref[...], preferred_element_type=f32) + bo_ref[...])

    full = lambda shape: pl.BlockSpec(shape, lambda i: tuple(0 for _ in shape))
    return pl.pallas_call(
        body,
        grid=(grid,),
        in_specs=[
            pl.BlockSpec((NC, BN, D), lambda i: (0, i, 0)),
            pl.BlockSpec((BN, D), lambda i: (i, 0)),
            pl.BlockSpec((BN, 1), lambda i: (i, 0)),
            pl.BlockSpec((BN, D), lambda i: (i, 0)),
            full((D, D)), full((1, D)),
            full((D, D)), full((1, D)),
            full((D, D)), full((1, D)),
            full((2 * D, D)), full((1, D)),
            full((2 * D, D)), full((1, D)),
            full((2 * D, D)), full((1, D)),
            full((D, 1)), full((1, 1)),
        ],
        out_specs=[
            pl.BlockSpec((BN, 1), lambda i: (i, 0)),
            pl.BlockSpec((BN, D), lambda i: (i, 0)),
        ],
        out_shape=[
            jax.ShapeDtypeStruct((N, 1), jnp.float32),
            jax.ShapeDtypeStruct((N, D), jnp.float32),
        ],
    )(zp, y, dis, h, Wz, bz2, Wr, br2, Wh, bh2, Lz, lbz2, Lr, lbr2, Lh, lbh2,
      Wo, bo2)


def kernel(x, edge_index, h, Wz, bz, Wr, br, Wh, bh,
           Lz, lbz, Lr, lbr, Lh, lbh, Wo, bo):
    ei = edge_index.astype(jnp.int32)
    srcs = ei[0]
    dsts = ei[1]

    ones_c = jnp.ones((CH, 1), jnp.float32)
    zeros_d = jnp.zeros((N, 1), jnp.float32)
    zeros_z = jnp.zeros((N, D), jnp.float32)

    dp = _degree_hist(dsts, ones_c, zeros_d)
    dis, y = _scale_rows(dp, x)
    zp = _aggregate(srcs, dsts, y, zeros_z)
    out, hn = _dense_cell(
        zp, y, dis, h,
        Wz, bz[None, :], Wr, br[None, :], Wh, bh[None, :],
        Lz, lbz[None, :], Lr, lbr[None, :], Lh, lbh[None, :],
        Wo, bo[None, :])
    return (out, hn)
